# Initial kernel scaffold; baseline (speedup 1.0000x reference)
import functools

import jax
import jax.numpy as jnp
from jax import lax
from jax.experimental import pallas as pl
from jax.experimental.pallas import tpu as pltpu
from jax.experimental.pallas import tpu_sc as plsc

GAMMA_ = 1.05

N_POINTS = 131072
N_BOXES = 256
K_SLOTS = 128
L = 16
NC = 2
NS = 16
NW = NC * NS
BOXES_PER_W = N_BOXES // NW
CHUNK = 4096
NCHUNKS = N_POINTS // CHUNK
VECS = CHUNK // L
ZROW = N_POINTS
TBL_ROWS = N_POINTS + 16
TBL_W = 8
IDXBUF = K_SLOTS + L


def _sc_body(x_hbm, y_hbm, cx_hbm, cy_hbm, t_hbm, tbl_hbm, out_hbm,
             xv, yv, cxv, cyv, tv, idxv, idxg, rows, sem):
    wid = lax.axis_index("s") * NC + lax.axis_index("c")

    pltpu.sync_copy(cx_hbm, cxv)
    pltpu.sync_copy(cy_hbm, cyv)
    pltpu.sync_copy(t_hbm, tv)

    lanes = lax.iota(jnp.int32, L)

    def do_box(i, carry):
        b = wid * BOXES_PER_W + i
        cx = cxv[b]
        cy = cyv[b]
        t = tv[b]

        zsplat = jnp.full((L,), ZROW, jnp.int32)

        def init(k, _c):
            idxv[pl.ds(k * L, L)] = zsplat
            return _c

        lax.fori_loop(0, IDXBUF // L, init, 0)

        def chunk_cond(cc):
            c, p = cc
            return jnp.logical_and(c < NCHUNKS, p < K_SLOTS)

        def chunk_body(cc):
            c, p = cc
            pltpu.sync_copy(x_hbm.at[pl.ds(c * CHUNK, CHUNK)], xv)
            pltpu.sync_copy(y_hbm.at[pl.ds(c * CHUNK, CHUNK)], yv)
            base = c * CHUNK

            def vec_cond(vc):
                j, q = vc
                return jnp.logical_and(j < VECS, q < K_SLOTS)

            def vec_body(vc):
                j, q = vc
                xs = xv[pl.ds(j * L, L)]
                ys = yv[pl.ds(j * L, L)]
                dx = xs - cx
                dy = ys - cy
                d2 = dx * dx + dy * dy
                m = d2 <= t
                cnt = plsc.all_reduce_population_count(m)[0]

                @pl.when(cnt > 0)
                def _():
                    idx = (base + j * L) + lanes
                    plsc.store_compressed(idxv.at[pl.ds(q, L)], idx, mask=m)

                return (j + 1, q + cnt)

            _, p = lax.while_loop(vec_cond, vec_body, (jnp.int32(0), p))
            return (c + 1, p)

        lax.while_loop(chunk_cond, chunk_body, (jnp.int32(0), jnp.int32(0)))

        def cp(k, _c):
            idxg[pl.ds(k * L, L)] = idxv[pl.ds(k * L, L)]
            return _c

        lax.fori_loop(0, K_SLOTS // L, cp, 0)

        pltpu.async_copy(tbl_hbm.at[idxg], rows, sem).wait()
        pltpu.sync_copy(rows, out_hbm.at[b])
        return carry

    lax.fori_loop(0, BOXES_PER_W, do_box, 0)


@functools.partial(
    pl.kernel,
    out_type=jax.ShapeDtypeStruct((N_BOXES, K_SLOTS, TBL_W), jnp.float32),
    mesh=plsc.VectorSubcoreMesh(core_axis_name="c", subcore_axis_name="s"),
    scratch_types=[
        pltpu.VMEM((CHUNK,), jnp.float32),
        pltpu.VMEM((CHUNK,), jnp.float32),
        pltpu.VMEM((N_BOXES,), jnp.float32),
        pltpu.VMEM((N_BOXES,), jnp.float32),
        pltpu.VMEM((N_BOXES,), jnp.float32),
        pltpu.VMEM((IDXBUF,), jnp.int32),
        pltpu.VMEM((K_SLOTS,), jnp.int32),
        pltpu.VMEM((K_SLOTS, TBL_W), jnp.float32),
        pltpu.SemaphoreType.DMA,
    ],
)
def _voxel_sample_sc(x_hbm, y_hbm, cx_hbm, cy_hbm, t_hbm, tbl_hbm, out_hbm,
                     *scratch):
    _sc_body(x_hbm, y_hbm, cx_hbm, cy_hbm, t_hbm, tbl_hbm, out_hbm, *scratch)


def _squared_threshold(r):
    t = r * r
    neg_inf = jnp.float32(-jnp.inf)
    pos_inf = jnp.float32(jnp.inf)
    for _ in range(8):
        t = jnp.where(jnp.sqrt(t) > r, jnp.nextafter(t, neg_inf), t)
    for _ in range(8):
        tn = jnp.nextafter(t, pos_inf)
        t = jnp.where(jnp.sqrt(tn) <= r, tn, t)
    return t


def kernel(cur_points, cur_boxes, num_sample):
    del num_sample
    pts = cur_points.astype(jnp.float32)
    x = pts[:, 0]
    y = pts[:, 1]
    r = jnp.linalg.norm(cur_boxes[:, 3:5] / 2.0, axis=-1) * GAMMA_
    t = _squared_threshold(r.astype(jnp.float32))
    cx = cur_boxes[:, 0].astype(jnp.float32)
    cy = cur_boxes[:, 1].astype(jnp.float32)
    tbl = jnp.zeros((TBL_ROWS, TBL_W), jnp.float32).at[:N_POINTS, :5].set(pts)
    out = _voxel_sample_sc(x, y, cx, cy, t, tbl)
    return out[:, :, :5]


# speedup vs baseline: 16.9093x; 16.9093x over previous
import functools

import jax
import jax.numpy as jnp
from jax import lax
from jax.experimental import pallas as pl
from jax.experimental.pallas import tpu as pltpu
from jax.experimental.pallas import tpu_sc as plsc

GAMMA_ = 1.05

N_POINTS = 131072
N_BOXES = 256
K_SLOTS = 128
L = 16
NC = 2
NS = 16
NW = NC * NS
BOXES_PER_W = N_BOXES // NW
CHUNK = 4096
NCHUNKS = N_POINTS // CHUNK
VECS = CHUNK // L
ZROW = N_POINTS
TBL_ROWS = N_POINTS + 16
TBL_W = 8
IDXBUF = K_SLOTS + L
PARAM_PAD = N_BOXES + L


def _sc_body(x_hbm, y_hbm, cx_hbm, cy_hbm, t_hbm, tbl_hbm, out_hbm,
             xv, yv, cxv, cyv, tv, idxv, idxg, rows, cnt_s, sem):
    wid = lax.axis_index("s") * NC + lax.axis_index("c")

    pltpu.sync_copy(cx_hbm, cxv)
    pltpu.sync_copy(cy_hbm, cyv)
    pltpu.sync_copy(t_hbm, tv)

    lanes = lax.iota(jnp.int32, L)

    pbase = wid * BOXES_PER_W
    cxvec = cxv[pl.ds(pbase, L)]
    cyvec = cyv[pl.ds(pbase, L)]
    tvec = tv[pl.ds(pbase, L)]

    def do_box(i, b):
        cx = cxvec[i]
        cy = cyvec[i]
        t = tvec[i]

        zsplat = jnp.full((L,), ZROW, jnp.int32)

        def init(k, _c):
            idxv[pl.ds(k * L, L)] = zsplat
            return _c

        lax.fori_loop(0, IDXBUF // L, init, 0)
        cnt_s[i] = jnp.int32(0)

        def chunk_body(c, _c):
            @pl.when(cnt_s[i] < K_SLOTS)
            def _():
                pltpu.sync_copy(x_hbm.at[pl.ds(c * CHUNK, CHUNK)], xv)
                pltpu.sync_copy(y_hbm.at[pl.ds(c * CHUNK, CHUNK)], yv)
                base = c * CHUNK

                def vec_body(j, q):
                    xs = xv[pl.ds(j * L, L)]
                    ys = yv[pl.ds(j * L, L)]
                    dx = xs - cx
                    dy = ys - cy
                    d2 = dx * dx + dy * dy
                    m = d2 <= t
                    cnt = plsc.all_reduce_population_count(m)[0]

                    @pl.when(jnp.logical_and(cnt > 0, q < K_SLOTS))
                    def _():
                        idx = (base + j * L) + lanes
                        plsc.store_compressed(idxv.at[pl.ds(q, L)], idx, mask=m)

                    return q + jnp.where(q < K_SLOTS, cnt, 0)

                q = lax.fori_loop(0, VECS, vec_body, cnt_s[i])
                cnt_s[i] = q

            return _c

        lax.fori_loop(0, NCHUNKS, chunk_body, 0)

        def cp(k, _c):
            idxg[pl.ds(k * L, L)] = idxv[pl.ds(k * L, L)]
            return _c

        lax.fori_loop(0, K_SLOTS // L, cp, 0)

        pltpu.async_copy(tbl_hbm.at[idxg], rows, sem).wait()
        pltpu.sync_copy(rows, out_hbm.at[b])

    for i in range(BOXES_PER_W):
        do_box(i, pbase + i)


@functools.partial(
    pl.kernel,
    out_type=jax.ShapeDtypeStruct((N_BOXES, K_SLOTS, TBL_W), jnp.float32),
    mesh=plsc.VectorSubcoreMesh(core_axis_name="c", subcore_axis_name="s"),
    scratch_types=[
        pltpu.VMEM((CHUNK,), jnp.float32),
        pltpu.VMEM((CHUNK,), jnp.float32),
        pltpu.VMEM((PARAM_PAD,), jnp.float32),
        pltpu.VMEM((PARAM_PAD,), jnp.float32),
        pltpu.VMEM((PARAM_PAD,), jnp.float32),
        pltpu.VMEM((IDXBUF,), jnp.int32),
        pltpu.VMEM((K_SLOTS,), jnp.int32),
        pltpu.VMEM((K_SLOTS, TBL_W), jnp.float32),
        pltpu.SMEM((BOXES_PER_W,), jnp.int32),
        pltpu.SemaphoreType.DMA,
    ],
    compiler_params=pltpu.CompilerParams(
        needs_layout_passes=False, use_tc_tiling_on_sc=False
    ),
)
def _voxel_sample_sc(x_hbm, y_hbm, cx_hbm, cy_hbm, t_hbm, tbl_hbm, out_hbm,
                     *scratch):
    _sc_body(x_hbm, y_hbm, cx_hbm, cy_hbm, t_hbm, tbl_hbm, out_hbm, *scratch)


def _squared_threshold(r):
    t = r * r
    neg_inf = jnp.float32(-jnp.inf)
    pos_inf = jnp.float32(jnp.inf)
    for _ in range(8):
        t = jnp.where(jnp.sqrt(t) > r, jnp.nextafter(t, neg_inf), t)
    for _ in range(8):
        tn = jnp.nextafter(t, pos_inf)
        t = jnp.where(jnp.sqrt(tn) <= r, tn, t)
    return t


def kernel(cur_points, cur_boxes, num_sample):
    del num_sample
    pts = cur_points.astype(jnp.float32)
    x = pts[:, 0]
    y = pts[:, 1]
    r = jnp.linalg.norm(cur_boxes[:, 3:5] / 2.0, axis=-1) * GAMMA_
    t = _squared_threshold(r.astype(jnp.float32))
    pad = jnp.zeros((PARAM_PAD - N_BOXES,), jnp.float32)
    cx = jnp.concatenate([cur_boxes[:, 0].astype(jnp.float32), pad])
    cy = jnp.concatenate([cur_boxes[:, 1].astype(jnp.float32), pad])
    t = jnp.concatenate([t.astype(jnp.float32), pad])
    tbl = jnp.zeros((TBL_ROWS, TBL_W), jnp.float32).at[:N_POINTS, :5].set(pts)
    out = _voxel_sample_sc(x, y, cx, cy, t, tbl)
    return out[:, :, :5]


# speedup vs baseline: 19.3825x; 1.1463x over previous
import functools

import jax
import jax.numpy as jnp
from jax import lax
from jax.experimental import pallas as pl
from jax.experimental.pallas import tpu as pltpu
from jax.experimental.pallas import tpu_sc as plsc

GAMMA_ = 1.05

N_POINTS = 131072
N_BOXES = 256
K_SLOTS = 128
L = 16
NC = 2
NS = 16
NW = NC * NS
BOXES_PER_W = N_BOXES // NW
CHUNK = 4096
NCHUNKS = N_POINTS // CHUNK
VECS = CHUNK // L
ZROW = N_POINTS
TBL_ROWS = N_POINTS + 16
TBL_W = 8
IDXBUF = K_SLOTS + L
PARAM_PAD = N_BOXES + L


def _sc_body(xy_hbm, cx_hbm, cy_hbm, t_hbm, tbl_hbm, out_hbm,
             buf0, buf1, cxv, cyv, tv, idxv, idxg, rows, cnt_s, infl_s,
             sem0, sem1):
    wid = lax.axis_index("s") * NC + lax.axis_index("c")

    pltpu.sync_copy(cx_hbm, cxv)
    pltpu.sync_copy(cy_hbm, cyv)
    pltpu.sync_copy(t_hbm, tv)

    lanes = lax.iota(jnp.int32, L)

    pbase = wid * BOXES_PER_W
    cxvec = cxv[pl.ds(pbase, L)]
    cyvec = cyv[pl.ds(pbase, L)]
    tvec = tv[pl.ds(pbase, L)]

    zsplat = jnp.full((L,), ZROW, jnp.int32)

    def init(k, _c):
        idxv[pl.ds(k * L, L)] = zsplat
        return _c

    lax.fori_loop(0, (BOXES_PER_W * IDXBUF) // L, init, 0)
    for i in range(BOXES_PER_W):
        cnt_s[i] = jnp.int32(0)

    bufs = (buf0, buf1)
    sems = (sem0, sem1)

    def start(c, parity):
        src = xy_hbm.at[pl.ds(c * (2 * CHUNK), 2 * CHUNK)]
        if parity == 0:
            pltpu.async_copy(src, bufs[0], sems[0])
        else:
            pltpu.async_copy(src, bufs[1], sems[1])

    def wait(c, parity):
        src = xy_hbm.at[pl.ds(c * (2 * CHUNK), 2 * CHUNK)]
        pltpu.make_async_copy(src, bufs[parity], sems[parity]).wait()

    start(jnp.int32(0), 0)
    infl_s[0] = jnp.int32(0)

    def chunk_step(c, parity):
        bufc = bufs[parity]
        done = cnt_s[0] >= K_SLOTS
        for i in range(1, BOXES_PER_W):
            done = jnp.logical_and(done, cnt_s[i] >= K_SLOTS)

        @pl.when(jnp.logical_and(jnp.logical_not(done), infl_s[0] == c))
        def _():
            @pl.when(c + 1 < NCHUNKS)
            def _():
                start(c + 1, 1 - parity)

            wait(c, parity)
            infl_s[0] = c + 1
            base = c * CHUNK

            for i in range(BOXES_PER_W):
                cx = cxvec[i]
                cy = cyvec[i]
                t = tvec[i]

                @pl.when(cnt_s[i] < K_SLOTS)
                def _(i=i, cx=cx, cy=cy, t=t):
                    def vcond(vc):
                        j, q = vc
                        return jnp.logical_and(j < VECS, q < K_SLOTS)

                    def vbody(vc):
                        j, q = vc
                        xs = bufc[pl.ds(j * L, L)]
                        ys = bufc[pl.ds(CHUNK + j * L, L)]
                        dx = xs - cx
                        dy = ys - cy
                        d2 = dx * dx + dy * dy
                        m = d2 <= t
                        idx = (base + j * L) + lanes
                        plsc.store_compressed(
                            idxv.at[pl.ds(i * IDXBUF + q, L)], idx, mask=m)
                        cnt = plsc.all_reduce_population_count(m)[0]
                        return (j + 1, q + cnt)

                    _, q = lax.while_loop(
                        vcond, vbody, (jnp.int32(0), cnt_s[i]))
                    cnt_s[i] = q

        @pl.when(jnp.logical_and(done, infl_s[0] == c))
        def _():
            wait(c, parity)
            infl_s[0] = jnp.int32(-1)

    def chunk_body(c2, _c):
        chunk_step(2 * c2, 0)
        chunk_step(2 * c2 + 1, 1)
        return _c

    lax.fori_loop(0, NCHUNKS // 2, chunk_body, 0)

    for i in range(BOXES_PER_W):
        def cp(k, _c, i=i):
            idxg[pl.ds(k * L, L)] = idxv[pl.ds(i * IDXBUF + k * L, L)]
            return _c

        lax.fori_loop(0, K_SLOTS // L, cp, 0)
        pltpu.async_copy(tbl_hbm.at[idxg], rows, sem0).wait()
        pltpu.sync_copy(rows, out_hbm.at[pbase + i])


@functools.partial(
    pl.kernel,
    out_type=jax.ShapeDtypeStruct((N_BOXES, K_SLOTS, TBL_W), jnp.float32),
    mesh=plsc.VectorSubcoreMesh(core_axis_name="c", subcore_axis_name="s"),
    scratch_types=[
        pltpu.VMEM((2 * CHUNK,), jnp.float32),
        pltpu.VMEM((2 * CHUNK,), jnp.float32),
        pltpu.VMEM((PARAM_PAD,), jnp.float32),
        pltpu.VMEM((PARAM_PAD,), jnp.float32),
        pltpu.VMEM((PARAM_PAD,), jnp.float32),
        pltpu.VMEM((BOXES_PER_W * IDXBUF,), jnp.int32),
        pltpu.VMEM((K_SLOTS,), jnp.int32),
        pltpu.VMEM((K_SLOTS, TBL_W), jnp.float32),
        pltpu.SMEM((BOXES_PER_W,), jnp.int32),
        pltpu.SMEM((1,), jnp.int32),
        pltpu.SemaphoreType.DMA,
        pltpu.SemaphoreType.DMA,
    ],
    compiler_params=pltpu.CompilerParams(
        needs_layout_passes=False, use_tc_tiling_on_sc=False
    ),
)
def _voxel_sample_sc(xy_hbm, cx_hbm, cy_hbm, t_hbm, tbl_hbm, out_hbm,
                     *scratch):
    _sc_body(xy_hbm, cx_hbm, cy_hbm, t_hbm, tbl_hbm, out_hbm, *scratch)


def _squared_threshold(r):
    t = r * r
    neg_inf = jnp.float32(-jnp.inf)
    pos_inf = jnp.float32(jnp.inf)
    for _ in range(8):
        t = jnp.where(jnp.sqrt(t) > r, jnp.nextafter(t, neg_inf), t)
    for _ in range(8):
        tn = jnp.nextafter(t, pos_inf)
        t = jnp.where(jnp.sqrt(tn) <= r, tn, t)
    return t


def kernel(cur_points, cur_boxes, num_sample):
    del num_sample
    pts = cur_points.astype(jnp.float32)
    xy = (
        pts[:, :2]
        .reshape(NCHUNKS, CHUNK, 2)
        .transpose(0, 2, 1)
        .reshape(NCHUNKS * 2 * CHUNK)
    )
    r = jnp.linalg.norm(cur_boxes[:, 3:5] / 2.0, axis=-1) * GAMMA_
    t = _squared_threshold(r.astype(jnp.float32))
    pad = jnp.zeros((PARAM_PAD - N_BOXES,), jnp.float32)
    cx = jnp.concatenate([cur_boxes[:, 0].astype(jnp.float32), pad])
    cy = jnp.concatenate([cur_boxes[:, 1].astype(jnp.float32), pad])
    t = jnp.concatenate([t.astype(jnp.float32), pad])
    tbl = jnp.zeros((TBL_ROWS, TBL_W), jnp.float32).at[:N_POINTS, :5].set(pts)
    out = _voxel_sample_sc(xy, cx, cy, t, tbl)
    return out[:, :, :5]


# speedup vs baseline: 24.0058x; 1.2385x over previous
import functools

import jax
import jax.numpy as jnp
from jax import lax
from jax.experimental import pallas as pl
from jax.experimental.pallas import tpu as pltpu
from jax.experimental.pallas import tpu_sc as plsc

GAMMA_ = 1.05

N_POINTS = 131072
N_BOXES = 256
K_SLOTS = 128
L = 16
NC = 2
NS = 16
NW = NC * NS
BOXES_PER_W = N_BOXES // NW
CHUNK = 8192
NCHUNKS = N_POINTS // CHUNK
BLOCK_VREGS = 8
BLOCK = BLOCK_VREGS * L
BLOCKS = CHUNK // BLOCK
ZROW = N_POINTS
TBL_ROWS = N_POINTS + 16
TBL_W = 8
IDXBUF = 2 * K_SLOTS
PARAM_PAD = N_BOXES + L


def _sc_body(xy_hbm, cx_hbm, cy_hbm, t_hbm, tbl_hbm, out_hbm,
             buf0, buf1, cxv, cyv, tv, idxv, idxg, rows, cnt_s, infl_s,
             sem0, sem1):
    wid = lax.axis_index("s") * NC + lax.axis_index("c")

    pltpu.sync_copy(cx_hbm, cxv)
    pltpu.sync_copy(cy_hbm, cyv)
    pltpu.sync_copy(t_hbm, tv)

    lanes = lax.iota(jnp.int32, L)

    pbase = wid * BOXES_PER_W
    cxvec = cxv[pl.ds(pbase, L)]
    cyvec = cyv[pl.ds(pbase, L)]
    tvec = tv[pl.ds(pbase, L)]

    zsplat = jnp.full((L,), ZROW, jnp.int32)

    def init(k, _c):
        idxv[pl.ds(k * L, L)] = zsplat
        return _c

    lax.fori_loop(0, (BOXES_PER_W * IDXBUF) // L, init, 0)
    for i in range(BOXES_PER_W):
        cnt_s[i] = jnp.int32(0)

    bufs = (buf0, buf1)
    sems = (sem0, sem1)

    def start(c, parity):
        src = xy_hbm.at[pl.ds(c * (2 * CHUNK), 2 * CHUNK)]
        if parity == 0:
            pltpu.async_copy(src, bufs[0], sems[0])
        else:
            pltpu.async_copy(src, bufs[1], sems[1])

    def wait(c, parity):
        src = xy_hbm.at[pl.ds(c * (2 * CHUNK), 2 * CHUNK)]
        pltpu.make_async_copy(src, bufs[parity], sems[parity]).wait()

    start(jnp.int32(0), 0)
    infl_s[0] = jnp.int32(0)

    def chunk_step(c, parity):
        bufc = bufs[parity]
        done = cnt_s[0] >= K_SLOTS
        for i in range(1, BOXES_PER_W):
            done = jnp.logical_and(done, cnt_s[i] >= K_SLOTS)

        @pl.when(jnp.logical_and(jnp.logical_not(done), infl_s[0] == c))
        def _():
            @pl.when(c + 1 < NCHUNKS)
            def _():
                start(c + 1, 1 - parity)

            wait(c, parity)
            infl_s[0] = c + 1
            base = c * CHUNK

            zero_v = jnp.zeros((L,), jnp.int32)
            one_v = jnp.full((L,), 1, jnp.int32)

            for i in range(BOXES_PER_W):
                cx = cxvec[i]
                cy = cyvec[i]
                t = tvec[i]

                @pl.when(cnt_s[i] < K_SLOTS)
                def _(i=i, cx=cx, cy=cy, t=t):
                    qstop = i * IDXBUF + (K_SLOTS - 1)

                    def vcond(vc):
                        blk, acc = vc
                        return jnp.logical_and(blk < BLOCKS, acc[0] < qstop)

                    def vbody(vc):
                        blk, acc = vc
                        for k in range(BLOCK_VREGS):
                            o = blk * BLOCK + k * L
                            xs = bufc[pl.ds(o, L)]
                            ys = bufc[pl.ds(CHUNK + o, L)]
                            dx = xs - cx
                            dy = ys - cy
                            d2 = dx * dx + dy * dy
                            m = d2 <= t
                            mi = jnp.where(m, one_v, zero_v)
                            s = plsc.cumsum(mi)
                            pos = acc + s
                            idx = (base + o) + lanes
                            plsc.store_scatter(idxv, [pos], idx, mask=m)
                            acc = acc + plsc.all_reduce_population_count(m)
                        return (blk + 1, acc)

                    q0 = cnt_s[i]
                    acc0 = zero_v + (i * IDXBUF - 1 + q0)
                    _, acc = lax.while_loop(
                        vcond, vbody, (jnp.int32(0), acc0))
                    cnt_s[i] = acc[0] - (i * IDXBUF - 1)

        @pl.when(jnp.logical_and(done, infl_s[0] == c))
        def _():
            wait(c, parity)
            infl_s[0] = jnp.int32(-1)

    def chunk_body(c2, _c):
        chunk_step(2 * c2, 0)
        chunk_step(2 * c2 + 1, 1)
        return _c

    lax.fori_loop(0, NCHUNKS // 2, chunk_body, 0)

    for i in range(BOXES_PER_W):
        def cp(k, _c, i=i):
            idxg[pl.ds(k * L, L)] = idxv[pl.ds(i * IDXBUF + k * L, L)]
            return _c

        lax.fori_loop(0, K_SLOTS // L, cp, 0)
        pltpu.async_copy(tbl_hbm.at[idxg], rows, sem0).wait()
        pltpu.sync_copy(rows, out_hbm.at[pbase + i])


@functools.partial(
    pl.kernel,
    out_type=jax.ShapeDtypeStruct((N_BOXES, K_SLOTS, TBL_W), jnp.float32),
    mesh=plsc.VectorSubcoreMesh(core_axis_name="c", subcore_axis_name="s"),
    scratch_types=[
        pltpu.VMEM((2 * CHUNK,), jnp.float32),
        pltpu.VMEM((2 * CHUNK,), jnp.float32),
        pltpu.VMEM((PARAM_PAD,), jnp.float32),
        pltpu.VMEM((PARAM_PAD,), jnp.float32),
        pltpu.VMEM((PARAM_PAD,), jnp.float32),
        pltpu.VMEM((BOXES_PER_W * IDXBUF,), jnp.int32),
        pltpu.VMEM((K_SLOTS,), jnp.int32),
        pltpu.VMEM((K_SLOTS, TBL_W), jnp.float32),
        pltpu.SMEM((BOXES_PER_W,), jnp.int32),
        pltpu.SMEM((1,), jnp.int32),
        pltpu.SemaphoreType.DMA,
        pltpu.SemaphoreType.DMA,
    ],
    compiler_params=pltpu.CompilerParams(
        needs_layout_passes=False, use_tc_tiling_on_sc=False
    ),
)
def _voxel_sample_sc(xy_hbm, cx_hbm, cy_hbm, t_hbm, tbl_hbm, out_hbm,
                     *scratch):
    _sc_body(xy_hbm, cx_hbm, cy_hbm, t_hbm, tbl_hbm, out_hbm, *scratch)


def _squared_threshold(r):
    t = r * r
    neg_inf = jnp.float32(-jnp.inf)
    pos_inf = jnp.float32(jnp.inf)
    for _ in range(8):
        t = jnp.where(jnp.sqrt(t) > r, jnp.nextafter(t, neg_inf), t)
    for _ in range(8):
        tn = jnp.nextafter(t, pos_inf)
        t = jnp.where(jnp.sqrt(tn) <= r, tn, t)
    return t


def kernel(cur_points, cur_boxes, num_sample):
    del num_sample
    pts = cur_points.astype(jnp.float32)
    xy = (
        pts[:, :2]
        .reshape(NCHUNKS, CHUNK, 2)
        .transpose(0, 2, 1)
        .reshape(NCHUNKS * 2 * CHUNK)
    )
    r = jnp.linalg.norm(cur_boxes[:, 3:5] / 2.0, axis=-1) * GAMMA_
    t = _squared_threshold(r.astype(jnp.float32))
    pad = jnp.zeros((PARAM_PAD - N_BOXES,), jnp.float32)
    cx = jnp.concatenate([cur_boxes[:, 0].astype(jnp.float32), pad])
    cy = jnp.concatenate([cur_boxes[:, 1].astype(jnp.float32), pad])
    t = jnp.concatenate([t.astype(jnp.float32), pad])
    tbl = jnp.zeros((TBL_ROWS, TBL_W), jnp.float32).at[:N_POINTS, :5].set(pts)
    out = _voxel_sample_sc(xy, cx, cy, t, tbl)
    return out[:, :, :5]
